# Initial kernel scaffold; baseline (speedup 1.0000x reference)
import jax
import jax.numpy as jnp
from jax.experimental import pallas as pl


def kernel(tag_ids, table, W1, b1, W2, b2, W3, b3):
    raise NotImplementedError("write your pallas kernel here")


# speedup vs baseline: 1.1437x; 1.1437x over previous
import functools

import jax
import jax.numpy as jnp
from jax import lax
from jax.experimental import pallas as pl
from jax.experimental.pallas import tpu as pltpu
from jax.experimental.pallas import tpu_sc as plsc

B = 16384
L = 200
D = 64
NUM_WORKERS = 32
ROWS_PER_WORKER = B // NUM_WORKERS
STAGE_ROWS = 64
NUM_STAGES = ROWS_PER_WORKER // STAGE_ROWS


def _sc_partial_sums(tag_ids, table):
    mesh = plsc.VectorSubcoreMesh(core_axis_name="c", subcore_axis_name="s")

    @functools.partial(
        pl.kernel,
        out_type=jax.ShapeDtypeStruct((NUM_WORKERS, L, D), jnp.float32),
        mesh=mesh,
        compiler_params=pltpu.CompilerParams(use_tc_tiling_on_sc=False),
        scratch_types=[
            pltpu.VMEM((STAGE_ROWS, L), jnp.int32),
            pltpu.VMEM((L, D), jnp.float32),
            pltpu.VMEM((L, D), jnp.float32),
            pltpu.SemaphoreType.DMA,
            pltpu.SemaphoreType.DMA,
        ],
    )
    def sc_kernel(ids_hbm, table_hbm, out_hbm, ids_v, rows_v, acc_v,
                  sem_ids, sem_g):
        wid = lax.axis_index("s") * 2 + lax.axis_index("c")
        row0 = wid * ROWS_PER_WORKER

        zeros = jnp.zeros((16,), jnp.float32)

        @pl.loop(0, L)
        def _zero(j):
            for d in range(D // 16):
                acc_v[j, pl.ds(d * 16, 16)] = zeros

        for s in range(NUM_STAGES):
            pltpu.async_copy(
                ids_hbm.at[pl.ds(row0 + s * STAGE_ROWS, STAGE_ROWS), :],
                ids_v, sem_ids).wait()

            @pl.loop(0, STAGE_ROWS)
            def _row(r):
                h1 = pltpu.async_copy(
                    table_hbm.at[ids_v.at[r, pl.ds(0, 128)]],
                    rows_v.at[pl.ds(0, 128), :], sem_g)
                h2 = pltpu.async_copy(
                    table_hbm.at[ids_v.at[r, pl.ds(128, L - 128)]],
                    rows_v.at[pl.ds(128, L - 128), :], sem_g)
                h1.wait()
                h2.wait()

                @pl.loop(0, L)
                def _accum(j):
                    for d in range(D // 16):
                        v = rows_v[j, pl.ds(d * 16, 16)]
                        plsc.addupdate(acc_v.at[j, pl.ds(d * 16, 16)], v)

        pltpu.sync_copy(acc_v, out_hbm.at[wid])

    return sc_kernel(tag_ids, table)


def _mlp(partials, W1, b1, W2, b2, W3, b3):
    def body(p_ref, w1_ref, b1_ref, w2_ref, b2_ref, w3_ref, b3_ref, o_ref):
        x = jnp.sum(p_ref[...], axis=0) * (1.0 / B)
        x = jnp.maximum(
            jnp.dot(x, w1_ref[...], preferred_element_type=jnp.float32)
            + b1_ref[...], 0.0)
        x = jnp.maximum(
            jnp.dot(x, w2_ref[...], preferred_element_type=jnp.float32)
            + b2_ref[...], 0.0)
        o_ref[...] = (
            jnp.dot(x, w3_ref[...], preferred_element_type=jnp.float32)
            + b3_ref[...])

    return pl.pallas_call(
        body,
        out_shape=jax.ShapeDtypeStruct((L, D), jnp.float32),
    )(partials, W1, b1.reshape(1, D), W2, b2.reshape(1, D),
      W3, b3.reshape(1, D))


def kernel(tag_ids, table, W1, b1, W2, b2, W3, b3):
    tag_ids = tag_ids.astype(jnp.int32)
    partials = _sc_partial_sums(tag_ids, table)
    return _mlp(partials, W1, b1, W2, b2, W3, b3)
